# baseline (speedup 1.0000x reference)
import functools

import jax
import jax.numpy as jnp
from jax.experimental import pallas as pl
from jax.experimental.pallas import tpu as pltpu


def _round_up(x, m):
    return (x + m - 1) // m * m


def _bigram_tile_kernel(idx_ref, tgt_ref, hi_ref, m_ref,
                        logits_ref, loss_ref, *, bb):
    v = hi_ref.shape[0]
    t = idx_ref.shape[1]

    idx = idx_ref[...]
    tgt = tgt_ref[...]
    row_iota = jax.lax.broadcasted_iota(jnp.int32, (v, t), 0)
    hi = hi_ref[...]

    counts = jnp.zeros((v, v), jnp.float32)
    for b in range(bb):
        oh_i = (row_iota == idx[b:b + 1, :]).astype(jnp.bfloat16)
        oh_t = (row_iota == tgt[b:b + 1, :]).astype(jnp.bfloat16)

        logits = jax.lax.dot_general(
            oh_i, hi, (((0,), (0,)), ((), ())),
            preferred_element_type=jnp.float32)
        logits_ref[b] = logits

        counts = counts + jax.lax.dot_general(
            oh_i, oh_t, (((1,), (1,)), ((), ())),
            preferred_element_type=jnp.float32)

    loss_ref[...] = jnp.broadcast_to(
        jnp.sum(counts * m_ref[...]), loss_ref.shape)


def kernel(idx, emb_table, targets, prepared_table, *, bb=48):
    B, T = idx.shape
    V = emb_table.shape[0]
    N = B * T

    table = prepared_table[:V, :V]
    lse = prepared_table[:V, V]
    hi = table.astype(jnp.bfloat16)
    m_loss = lse[:, None] - table

    idx = idx.astype(jnp.int32)
    tgt = targets.astype(jnp.int32)
    B_pad = _round_up(B, bb)
    T_pad = _round_up(T, 128)
    if B_pad != B or T_pad != T:
        pad = ((0, B_pad - B), (0, T_pad - T))
        idx = jnp.pad(idx, pad, constant_values=V)
        tgt = jnp.pad(tgt, pad, constant_values=V)
    num_tiles = B_pad // bb

    logits_p, partials = pl.pallas_call(
        functools.partial(_bigram_tile_kernel, bb=bb),
        out_shape=(
            jax.ShapeDtypeStruct((B_pad, T_pad, V), jnp.float32),
            jax.ShapeDtypeStruct((num_tiles, 1, 128), jnp.float32),
        ),
        grid=(num_tiles,),
        in_specs=[
            pl.BlockSpec((bb, T_pad), lambda i: (i, 0)),
            pl.BlockSpec((bb, T_pad), lambda i: (i, 0)),
            pl.BlockSpec((V, V), lambda i: (0, 0)),
            pl.BlockSpec((V, V), lambda i: (0, 0)),
        ],
        out_specs=(
            pl.BlockSpec((bb, T_pad, V), lambda i: (i, 0, 0)),
            pl.BlockSpec((1, 1, 128), lambda i: (i, 0, 0)),
        ),
        compiler_params=pltpu.CompilerParams(
            dimension_semantics=("parallel",),
            vmem_limit_bytes=58 * 1024 * 1024,
        ),
    )(idx, tgt, hi, m_loss)

    loss = jnp.sum(partials[:, 0, 0]) / N
    if B_pad != B or T_pad != T:
        logits = logits_p[:B, :T].reshape(N, V)
    else:
        logits = logits_p.reshape(N, V)
    return logits, loss


# speedup vs baseline: 1.0013x; 1.0013x over previous
import functools

import jax
import jax.numpy as jnp
from jax.experimental import pallas as pl
from jax.experimental.pallas import tpu as pltpu


def _round_up(x, m):
    return (x + m - 1) // m * m


def _bigram_tile_kernel(idx_ref, tgt_ref, bd_ref, m2_ref,
                        logits_ref, loss_ref, *, bb):
    v = bd_ref.shape[0] // 2
    t = idx_ref.shape[1]

    idx = idx_ref[...]
    tgt = tgt_ref[...]
    iota2 = jax.lax.broadcasted_iota(jnp.int32, (2 * v, t), 0)
    upper = iota2 >= v
    bd = bd_ref[...]

    counts = jnp.zeros((2 * v, 2 * v), jnp.float32)
    for b in range(0, bb, 2):
        key_i = jnp.where(upper, idx[b + 1:b + 2, :] + v, idx[b:b + 1, :])
        key_t = jnp.where(upper, tgt[b + 1:b + 2, :] + v, tgt[b:b + 1, :])
        oh_i = (iota2 == key_i).astype(jnp.bfloat16)
        oh_t = (iota2 == key_t).astype(jnp.bfloat16)

        pair = jax.lax.dot_general(
            oh_i, bd, (((0,), (0,)), ((), ())),
            preferred_element_type=jnp.float32)
        logits_ref[b] = pair[:, :v]
        logits_ref[b + 1] = pair[:, v:]

        counts = counts + jax.lax.dot_general(
            oh_i, oh_t, (((1,), (1,)), ((), ())),
            preferred_element_type=jnp.float32)

    loss_ref[...] = jnp.broadcast_to(
        jnp.sum(counts * m2_ref[...]), loss_ref.shape)


def kernel(idx, emb_table, targets, prepared_table, *, bb=48):
    B, T = idx.shape
    V = emb_table.shape[0]
    N = B * T

    table = prepared_table[:V, :V]
    lse = prepared_table[:V, V]
    hi = table.astype(jnp.bfloat16)
    zero = jnp.zeros((V, V), jnp.bfloat16)
    bd = jnp.block([[hi, zero], [zero, hi]])
    m_loss = lse[:, None] - table
    zf = jnp.zeros((V, V), jnp.float32)
    m2 = jnp.block([[m_loss, zf], [zf, m_loss]])

    idx = idx.astype(jnp.int32)
    tgt = targets.astype(jnp.int32)
    B_pad = _round_up(B, bb)
    T_pad = _round_up(T, 128)
    if B_pad != B or T_pad != T:
        pad = ((0, B_pad - B), (0, T_pad - T))
        idx = jnp.pad(idx, pad, constant_values=V)
        tgt = jnp.pad(tgt, pad, constant_values=V)
    num_tiles = B_pad // bb

    logits_p, partials = pl.pallas_call(
        functools.partial(_bigram_tile_kernel, bb=bb),
        out_shape=(
            jax.ShapeDtypeStruct((B_pad, T_pad, V), jnp.float32),
            jax.ShapeDtypeStruct((num_tiles, 1, 128), jnp.float32),
        ),
        grid=(num_tiles,),
        in_specs=[
            pl.BlockSpec((bb, T_pad), lambda i: (i, 0)),
            pl.BlockSpec((bb, T_pad), lambda i: (i, 0)),
            pl.BlockSpec((2 * V, 2 * V), lambda i: (0, 0)),
            pl.BlockSpec((2 * V, 2 * V), lambda i: (0, 0)),
        ],
        out_specs=(
            pl.BlockSpec((bb, T_pad, V), lambda i: (i, 0, 0)),
            pl.BlockSpec((1, 1, 128), lambda i: (i, 0, 0)),
        ),
        compiler_params=pltpu.CompilerParams(
            dimension_semantics=("parallel",),
            vmem_limit_bytes=58 * 1024 * 1024,
        ),
    )(idx, tgt, bd, m2)

    loss = jnp.sum(partials[:, 0, 0]) / N
    if B_pad != B or T_pad != T:
        logits = logits_p[:B, :T].reshape(N, V)
    else:
        logits = logits_p.reshape(N, V)
    return logits, loss
